# baseline (speedup 1.0000x reference)
import functools

import jax
import jax.numpy as jnp
from jax import lax
from jax.experimental import pallas as pl
from jax.experimental.pallas import tpu as pltpu
from jax.experimental.pallas import tpu_sc as plsc

N = 10000
E = 320000

NC = 2
NS = 16
NW = NC * NS
CH = 128
CG = 8
NCHUNKS = 2560
TPT = NCHUNKS // NS
E_PAD = NCHUNKS * CH
N_PAD = 10112
RPT = N_PAD // NS


_MESH = plsc.VectorSubcoreMesh(
    core_axis_name="c", subcore_axis_name="s", num_cores=NC)


def _gather_loop(table, src_r, dst_r, srcv, dstv, bufs, sems, acc, chunk0):
    def group(g, carry):
        g0 = pl.multiple_of(chunk0 + g * CG, CG)
        pltpu.sync_copy(src_r.at[pl.ds(g0, CG)], srcv)
        pltpu.sync_copy(dst_r.at[pl.ds(g0, CG)], dstv)
        h = pltpu.async_copy(table.at[srcv.at[0]], bufs[0], sems[0])
        for j in range(CG):
            if j + 1 < CG:
                h_next = pltpu.async_copy(
                    table.at[srcv.at[j + 1]], bufs[(j + 1) % 2],
                    sems[(j + 1) % 2])
            h.wait()
            pltpu.sync_copy(bufs[j % 2], acc.at[dstv.at[j]], add=True)
            if j + 1 < CG:
                h = h_next
        return carry

    lax.fori_loop(0, TPT // CG, group, 0)


def _make_agg(with_cnt):
    out_type = [jax.ShapeDtypeStruct((N_PAD, 128), jnp.float32)]
    scratch = [
        pltpu.VMEM((CG, CH), jnp.int32),
        pltpu.VMEM((CG, CH), jnp.int32),
        pltpu.VMEM((CH, 128), jnp.float32),
        pltpu.VMEM((CH, 128), jnp.float32),
        pltpu.VMEM_SHARED((N_PAD, 128), jnp.float32),
        pltpu.SemaphoreType.DMA,
        pltpu.SemaphoreType.DMA,
    ]
    if with_cnt:
        out_type.append(jax.ShapeDtypeStruct((N_PAD, 128), jnp.float32))

    def common(table, src_r, dst_r, zeros, out_s, srcv, dstv, rows0, rows1,
               acc, sem0, sem1, out_c=None):
        c = lax.axis_index("c")
        s = lax.axis_index("s")
        r0 = s * RPT
        chunk0 = s * TPT
        if with_cnt:
            pltpu.sync_copy(zeros.at[pl.ds(r0, RPT)], acc.at[pl.ds(r0, RPT)])
        else:
            @pl.when(c == 0)
            def _():
                pltpu.sync_copy(zeros.at[pl.ds(r0, RPT)],
                                acc.at[pl.ds(r0, RPT)])
        plsc.subcore_barrier()

        @pl.when(c == 0)
        def _():
            _gather_loop(table, src_r, dst_r, srcv, dstv, (rows0, rows1),
                         (sem0, sem1), acc, chunk0)

        if with_cnt:
            @pl.when(c == 1)
            def _():
                def group(g, carry):
                    g0 = pl.multiple_of(chunk0 + g * CG, CG)
                    pltpu.sync_copy(dst_r.at[pl.ds(g0, CG)], dstv)
                    for j in range(CG):
                        pltpu.sync_copy(rows0, acc.at[dstv.at[j]], add=True)
                    return carry

                lax.fori_loop(0, TPT // CG, group, 0)

        plsc.subcore_barrier()

        @pl.when(c == 0)
        def _():
            pltpu.sync_copy(acc.at[pl.ds(r0, RPT)], out_s.at[pl.ds(r0, RPT)])

        if with_cnt:
            @pl.when(c == 1)
            def _():
                pltpu.sync_copy(acc.at[pl.ds(r0, RPT)],
                                out_c.at[pl.ds(r0, RPT)])

    if with_cnt:
        def body(table, src_r, dst_r, zeros, ones_h, out_s, out_c,
                 srcv, dstv, rows0, rows1, acc, sem0, sem1):
            c = lax.axis_index("c")

            @pl.when(c == 1)
            def _():
                pltpu.sync_copy(ones_h, rows0)

            common(table, src_r, dst_r, zeros, out_s, srcv, dstv, rows0,
                   rows1, acc, sem0, sem1, out_c=out_c)
    else:
        def body(table, src_r, dst_r, zeros, out_s,
                 srcv, dstv, rows0, rows1, acc, sem0, sem1):
            common(table, src_r, dst_r, zeros, out_s, srcv, dstv, rows0,
                   rows1, acc, sem0, sem1)

    return pl.kernel(body, mesh=_MESH, out_type=out_type,
                     scratch_types=scratch)


_agg_cnt = _make_agg(True)
_agg128 = _make_agg(False)

B = 1000
GRID = N // B


def _inv_cnt(cntp_ref):
    return 1.0 / jnp.maximum(cntp_ref[:, 0:1], 1.0)


def _tc_a(s1p, cntp, x, w1lt, b1, w1rt, w2lt, h1_o, y2_o):
    mean = s1p[...] * _inv_cnt(cntp)
    h1 = jnp.maximum(
        jnp.dot(mean, w1lt[...], preferred_element_type=jnp.float32)
        + b1[...]
        + jnp.dot(x[...], w1rt[...], preferred_element_type=jnp.float32),
        0.0,
    )
    h1_o[...] = h1
    y2_o[...] = jnp.dot(h1, w2lt[...], preferred_element_type=jnp.float32)


def _tc_b(s2p, cntp, h1, w2rt, b2, wlint, blin, h3_o):
    mean2 = s2p[...] * _inv_cnt(cntp)
    h2 = jnp.maximum(
        mean2 + b2[...]
        + jnp.dot(h1[...], w2rt[...], preferred_element_type=jnp.float32),
        0.0,
    )
    h3_o[...] = (
        jnp.dot(h2, wlint[...], preferred_element_type=jnp.float32) + blin[...]
    )


def _tc_c(s4p, cntp, h3, w4lt, b4, w4rt, out_o):
    mean4 = s4p[...] * _inv_cnt(cntp)
    out_o[...] = (
        jnp.dot(mean4, w4lt[...], preferred_element_type=jnp.float32)
        + b4[...]
        + jnp.dot(h3[...], w4rt[...], preferred_element_type=jnp.float32)
    )


def _rows(d):
    return pl.BlockSpec((B, d), lambda i: (i, 0))


def _part(d):
    return pl.BlockSpec((B, d), lambda i: (i, 0))


def _full(r, c):
    return pl.BlockSpec((r, c), lambda i: (0, 0))


def kernel(x, edge_index, W1l, b1, W1r, W2l, b2, W2r, Wlin, blin, W4l, b4, W4r):
    src = jnp.concatenate(
        [edge_index[0], jnp.zeros((E_PAD - E,), jnp.int32)]).reshape(NCHUNKS, CH)
    pad_dst = N + jnp.arange(E_PAD - E, dtype=jnp.int32) % (N_PAD - N)
    dst = jnp.concatenate([edge_index[1], pad_dst]).reshape(NCHUNKS, CH)
    z128 = jnp.zeros((N_PAD, 128), jnp.float32)
    ones128 = jnp.ones((CH, 128), jnp.float32)

    s1p, cntp = _agg_cnt(x, src, dst, z128, ones128)

    h1, y2 = pl.pallas_call(
        _tc_a,
        grid=(GRID,),
        in_specs=[_part(128), _part(128), _rows(128), _full(128, 256),
                  _full(1, 256), _full(128, 256), _full(256, 128)],
        out_specs=[_rows(256), _rows(128)],
        out_shape=[jax.ShapeDtypeStruct((N, 256), jnp.float32),
                   jax.ShapeDtypeStruct((N, 128), jnp.float32)],
    )(s1p, cntp, x, W1l.T, b1.reshape(1, -1), W1r.T, W2l.T)

    (s2p,) = _agg128(y2, src, dst, z128)

    h3 = pl.pallas_call(
        _tc_b,
        grid=(GRID,),
        in_specs=[_part(128), _part(128), _rows(256), _full(256, 128),
                  _full(1, 128), _full(128, 128), _full(1, 128)],
        out_specs=_rows(128),
        out_shape=jax.ShapeDtypeStruct((N, 128), jnp.float32),
    )(s2p, cntp, h1, W2r.T, b2.reshape(1, -1), Wlin.T, blin.reshape(1, -1))

    (s4p,) = _agg128(h3, src, dst, z128)

    out = pl.pallas_call(
        _tc_c,
        grid=(GRID,),
        in_specs=[_part(128), _part(128), _rows(128), _full(128, 64),
                  _full(1, 64), _full(128, 64)],
        out_specs=_rows(64),
        out_shape=jax.ShapeDtypeStruct((N, 64), jnp.float32),
    )(s4p, cntp, h3, W4l.T, b4.reshape(1, -1), W4r.T)

    return out


# speedup vs baseline: 2.4390x; 2.4390x over previous
import functools

import jax
import jax.numpy as jnp
from jax import lax
from jax.experimental import pallas as pl
from jax.experimental.pallas import tpu as pltpu
from jax.experimental.pallas import tpu_sc as plsc

N = 10000
E = 320000

NC = 2
NS = 16
NW = NC * NS
CH = 128
CG = 8
NCHUNKS = 2560
TPT = NCHUNKS // NS
E_PAD = NCHUNKS * CH
N_PAD = 10112
RPT = N_PAD // NS


_MESH = plsc.VectorSubcoreMesh(
    core_axis_name="c", subcore_axis_name="s", num_cores=NC)


def _gather_loop(table, src_r, dst_r, srcv, dstv, bufs, sems, acc, chunk0):
    def group(g, carry):
        g0 = pl.multiple_of(chunk0 + g * CG, CG)
        pltpu.sync_copy(src_r.at[pl.ds(g0, CG)], srcv)
        pltpu.sync_copy(dst_r.at[pl.ds(g0, CG)], dstv)
        h = pltpu.async_copy(table.at[srcv.at[0]], bufs[0], sems[0])
        for j in range(CG):
            if j + 1 < CG:
                h_next = pltpu.async_copy(
                    table.at[srcv.at[j + 1]], bufs[(j + 1) % 2],
                    sems[(j + 1) % 2])
            h.wait()
            pltpu.sync_copy(bufs[j % 2], acc.at[dstv.at[j]], add=True)
            if j + 1 < CG:
                h = h_next
        return carry

    lax.fori_loop(0, TPT // CG, group, 0)


def _make_agg(with_cnt):
    out_type = [jax.ShapeDtypeStruct((N_PAD, 128), jnp.float32)]
    scratch = [
        pltpu.VMEM((CG, CH), jnp.int32),
        pltpu.VMEM((CG, CH), jnp.int32),
        pltpu.VMEM((CH, 128), jnp.float32),
        pltpu.VMEM((CH, 128), jnp.float32),
        pltpu.VMEM_SHARED((N_PAD, 128), jnp.float32),
        pltpu.SemaphoreType.DMA,
        pltpu.SemaphoreType.DMA,
    ]
    if with_cnt:
        out_type.append(jax.ShapeDtypeStruct((N_PAD, 128), jnp.float32))

    def common(table, src_r, dst_r, zeros, out_s, srcv, dstv, rows0, rows1,
               acc, sem0, sem1, out_c=None):
        c = lax.axis_index("c")
        s = lax.axis_index("s")
        r0 = s * RPT
        chunk0 = s * TPT
        if with_cnt:
            pltpu.sync_copy(zeros.at[pl.ds(r0, RPT)], acc.at[pl.ds(r0, RPT)])
        else:
            @pl.when(c == 0)
            def _():
                pltpu.sync_copy(zeros.at[pl.ds(r0, RPT)],
                                acc.at[pl.ds(r0, RPT)])
        plsc.subcore_barrier()

        @pl.when(c == 0)
        def _():
            _gather_loop(table, src_r, dst_r, srcv, dstv, (rows0, rows1),
                         (sem0, sem1), acc, chunk0)

        if with_cnt:
            @pl.when(c == 1)
            def _():
                def group(g, carry):
                    g0 = pl.multiple_of(chunk0 + g * CG, CG)
                    pltpu.sync_copy(dst_r.at[pl.ds(g0, CG)], dstv)
                    for j in range(CG):
                        pltpu.sync_copy(rows0, acc.at[dstv.at[j]], add=True)
                    return carry

                lax.fori_loop(0, TPT // CG, group, 0)

        plsc.subcore_barrier()

        @pl.when(c == 0)
        def _():
            pltpu.sync_copy(acc.at[pl.ds(r0, RPT)], out_s.at[pl.ds(r0, RPT)])

        if with_cnt:
            @pl.when(c == 1)
            def _():
                pltpu.sync_copy(acc.at[pl.ds(r0, RPT)],
                                out_c.at[pl.ds(r0, RPT)])

    if with_cnt:
        def body(table, src_r, dst_r, zeros, ones_h, out_s, out_c,
                 srcv, dstv, rows0, rows1, acc, sem0, sem1):
            c = lax.axis_index("c")

            @pl.when(c == 1)
            def _():
                pltpu.sync_copy(ones_h, rows0)

            common(table, src_r, dst_r, zeros, out_s, srcv, dstv, rows0,
                   rows1, acc, sem0, sem1, out_c=out_c)
    else:
        def body(table, src_r, dst_r, zeros, out_s,
                 srcv, dstv, rows0, rows1, acc, sem0, sem1):
            common(table, src_r, dst_r, zeros, out_s, srcv, dstv, rows0,
                   rows1, acc, sem0, sem1)

    return pl.kernel(body, mesh=_MESH, out_type=out_type,
                     scratch_types=scratch)


_agg_cnt = _make_agg(True)
_agg128 = _make_agg(False)

B = 1000
GRID = N // B


def _inv_cnt(cntp_ref):
    return 1.0 / jnp.maximum(cntp_ref[:, 0:1], 1.0)


def _tc_a(s1p, cntp, x, w1lt, b1, w1rt, w2lt, h1_o, y2_o):
    mean = s1p[...] * _inv_cnt(cntp)
    h1 = jnp.maximum(
        jnp.dot(mean, w1lt[...], preferred_element_type=jnp.float32)
        + b1[...]
        + jnp.dot(x[...], w1rt[...], preferred_element_type=jnp.float32),
        0.0,
    )
    h1_o[...] = h1
    y2_o[...] = jnp.dot(h1, w2lt[...], preferred_element_type=jnp.float32)


def _tc_b(s2p, cntp, h1, w2rt, b2, wlint, blin, h3_o):
    mean2 = s2p[...] * _inv_cnt(cntp)
    h2 = jnp.maximum(
        mean2 + b2[...]
        + jnp.dot(h1[...], w2rt[...], preferred_element_type=jnp.float32),
        0.0,
    )
    h3_o[...] = (
        jnp.dot(h2, wlint[...], preferred_element_type=jnp.float32) + blin[...]
    )


def _tc_c(s4p, cntp, h3, w4lt, b4, w4rt, out_o):
    mean4 = s4p[...] * _inv_cnt(cntp)
    out_o[...] = (
        jnp.dot(mean4, w4lt[...], preferred_element_type=jnp.float32)
        + b4[...]
        + jnp.dot(h3[...], w4rt[...], preferred_element_type=jnp.float32)
    )


def _rows(d):
    return pl.BlockSpec((B, d), lambda i: (i, 0))


def _part(d):
    return pl.BlockSpec((B, d), lambda i: (i, 0))


def _full(r, c):
    return pl.BlockSpec((r, c), lambda i: (0, 0))


def kernel(x, edge_index, W1l, b1, W1r, W2l, b2, W2r, Wlin, blin, W4l, b4, W4r):
    pad_src = jnp.arange(E_PAD - E, dtype=jnp.int32) % N
    src = jnp.concatenate([edge_index[0], pad_src]).reshape(NCHUNKS, CH)
    pad_dst = N + jnp.arange(E_PAD - E, dtype=jnp.int32) % (N_PAD - N)
    dst = jnp.concatenate([edge_index[1], pad_dst]).reshape(NCHUNKS, CH)
    z128 = jnp.zeros((N_PAD, 128), jnp.float32)
    ones128 = jnp.ones((CH, 128), jnp.float32)

    s1p, cntp = _agg_cnt(x, src, dst, z128, ones128)

    h1, y2 = pl.pallas_call(
        _tc_a,
        grid=(GRID,),
        in_specs=[_part(128), _part(128), _rows(128), _full(128, 256),
                  _full(1, 256), _full(128, 256), _full(256, 128)],
        out_specs=[_rows(256), _rows(128)],
        out_shape=[jax.ShapeDtypeStruct((N, 256), jnp.float32),
                   jax.ShapeDtypeStruct((N, 128), jnp.float32)],
    )(s1p, cntp, x, W1l.T, b1.reshape(1, -1), W1r.T, W2l.T)

    (s2p,) = _agg128(y2, src, dst, z128)

    h3 = pl.pallas_call(
        _tc_b,
        grid=(GRID,),
        in_specs=[_part(128), _part(128), _rows(256), _full(256, 128),
                  _full(1, 128), _full(128, 128), _full(1, 128)],
        out_specs=_rows(128),
        out_shape=jax.ShapeDtypeStruct((N, 128), jnp.float32),
    )(s2p, cntp, h1, W2r.T, b2.reshape(1, -1), Wlin.T, blin.reshape(1, -1))

    (s4p,) = _agg128(h3, src, dst, z128)

    out = pl.pallas_call(
        _tc_c,
        grid=(GRID,),
        in_specs=[_part(128), _part(128), _rows(128), _full(128, 64),
                  _full(1, 64), _full(128, 64)],
        out_specs=_rows(64),
        out_shape=jax.ShapeDtypeStruct((N, 64), jnp.float32),
    )(s4p, cntp, h3, W4l.T, b4.reshape(1, -1), W4r.T)

    return out


# speedup vs baseline: 3.3399x; 1.3694x over previous
import functools

import jax
import jax.numpy as jnp
from jax import lax
from jax.experimental import pallas as pl
from jax.experimental.pallas import tpu as pltpu
from jax.experimental.pallas import tpu_sc as plsc

N = 10000
E = 320000

NC = 2
NS = 16
NW = NC * NS
CH = 128
CG = 8
NCHUNKS = 2560
TPT = NCHUNKS // NS
E_PAD = NCHUNKS * CH
N_PAD = 10112
RPT = N_PAD // NS


_MESH = plsc.VectorSubcoreMesh(
    core_axis_name="c", subcore_axis_name="s", num_cores=NC)


def _gather_loop(table, src_r, dst_r, srcv, dstv, bufs, sems, acc, chunk0,
                 ngroups):
    def group(g, carry):
        g0 = pl.multiple_of(chunk0 + g * CG, CG)
        pltpu.sync_copy(src_r.at[pl.ds(g0, CG)], srcv)
        pltpu.sync_copy(dst_r.at[pl.ds(g0, CG)], dstv)
        h = pltpu.async_copy(table.at[srcv.at[0]], bufs[0], sems[0])
        for j in range(CG):
            if j + 1 < CG:
                h_next = pltpu.async_copy(
                    table.at[srcv.at[j + 1]], bufs[(j + 1) % 2],
                    sems[(j + 1) % 2])
            h.wait()
            pltpu.sync_copy(bufs[j % 2], acc.at[dstv.at[j]], add=True)
            if j + 1 < CG:
                h = h_next
        return carry

    lax.fori_loop(0, ngroups, group, 0)


def _make_agg(with_cnt):
    out_type = [jax.ShapeDtypeStruct((N_PAD, 128), jnp.float32)]
    scratch = [
        pltpu.VMEM((CG, CH), jnp.int32),
        pltpu.VMEM((CG, CH), jnp.int32),
        pltpu.VMEM((CH, 128), jnp.float32),
        pltpu.VMEM((CH, 128), jnp.float32),
        pltpu.VMEM_SHARED((N_PAD, 128), jnp.float32),
        pltpu.SemaphoreType.DMA,
        pltpu.SemaphoreType.DMA,
    ]
    if with_cnt:
        out_type.append(jax.ShapeDtypeStruct((N_PAD, 128), jnp.float32))
    else:
        out_type = [jax.ShapeDtypeStruct((NC, N_PAD, 128), jnp.float32)]

    def common(table, src_r, dst_r, zeros, out_s, srcv, dstv, rows0, rows1,
               acc, sem0, sem1, out_c=None):
        c = lax.axis_index("c")
        s = lax.axis_index("s")
        r0 = s * RPT
        pltpu.sync_copy(zeros.at[pl.ds(r0, RPT)], acc.at[pl.ds(r0, RPT)])
        plsc.subcore_barrier()

        if with_cnt:
            chunk0 = s * TPT

            @pl.when(c == 0)
            def _():
                _gather_loop(table, src_r, dst_r, srcv, dstv, (rows0, rows1),
                             (sem0, sem1), acc, chunk0, TPT // CG)

            @pl.when(c == 1)
            def _():
                def group(g, carry):
                    g0 = pl.multiple_of(chunk0 + g * CG, CG)
                    pltpu.sync_copy(dst_r.at[pl.ds(g0, CG)], dstv)
                    for j in range(CG):
                        pltpu.sync_copy(rows0, acc.at[dstv.at[j]], add=True)
                    return carry

                lax.fori_loop(0, TPT // CG, group, 0)
        else:
            chunk0 = c * (NCHUNKS // NC) + s * (TPT // NC)
            _gather_loop(table, src_r, dst_r, srcv, dstv, (rows0, rows1),
                         (sem0, sem1), acc, chunk0, TPT // NC // CG)

        plsc.subcore_barrier()

        if with_cnt:
            @pl.when(c == 0)
            def _():
                pltpu.sync_copy(acc.at[pl.ds(r0, RPT)],
                                out_s.at[pl.ds(r0, RPT)])

            @pl.when(c == 1)
            def _():
                pltpu.sync_copy(acc.at[pl.ds(r0, RPT)],
                                out_c.at[pl.ds(r0, RPT)])
        else:
            pltpu.sync_copy(acc.at[pl.ds(r0, RPT)],
                            out_s.at[c, pl.ds(r0, RPT)])

    if with_cnt:
        def body(table, src_r, dst_r, zeros, ones_h, out_s, out_c,
                 srcv, dstv, rows0, rows1, acc, sem0, sem1):
            c = lax.axis_index("c")

            @pl.when(c == 1)
            def _():
                pltpu.sync_copy(ones_h, rows0)

            common(table, src_r, dst_r, zeros, out_s, srcv, dstv, rows0,
                   rows1, acc, sem0, sem1, out_c=out_c)
    else:
        def body(table, src_r, dst_r, zeros, out_s,
                 srcv, dstv, rows0, rows1, acc, sem0, sem1):
            common(table, src_r, dst_r, zeros, out_s, srcv, dstv, rows0,
                   rows1, acc, sem0, sem1)

    return pl.kernel(body, mesh=_MESH, out_type=out_type,
                     scratch_types=scratch)


_agg_cnt = _make_agg(True)
_agg128 = _make_agg(False)

B = 1000
GRID = N // B


def _inv_cnt(cntp_ref):
    return 1.0 / jnp.maximum(cntp_ref[:, 0:1], 1.0)


def _tc_a(s1p, cntp, x, w1lt, b1, w1rt, w2lt, h1_o, y2_o):
    mean = s1p[...] * _inv_cnt(cntp)
    h1 = jnp.maximum(
        jnp.dot(mean, w1lt[...], preferred_element_type=jnp.float32)
        + b1[...]
        + jnp.dot(x[...], w1rt[...], preferred_element_type=jnp.float32),
        0.0,
    )
    h1_o[...] = h1
    y2_o[...] = jnp.dot(h1, w2lt[...], preferred_element_type=jnp.float32)


def _tc_b(s2p, cntp, h1, w2rt, b2, wlint, blin, h3_o):
    mean2 = (s2p[0] + s2p[1]) * _inv_cnt(cntp)
    h2 = jnp.maximum(
        mean2 + b2[...]
        + jnp.dot(h1[...], w2rt[...], preferred_element_type=jnp.float32),
        0.0,
    )
    h3_o[...] = (
        jnp.dot(h2, wlint[...], preferred_element_type=jnp.float32) + blin[...]
    )


def _tc_c(s4p, cntp, h3, w4lt, b4, w4rt, out_o):
    mean4 = (s4p[0] + s4p[1]) * _inv_cnt(cntp)
    out_o[...] = (
        jnp.dot(mean4, w4lt[...], preferred_element_type=jnp.float32)
        + b4[...]
        + jnp.dot(h3[...], w4rt[...], preferred_element_type=jnp.float32)
    )


def _rows(d):
    return pl.BlockSpec((B, d), lambda i: (i, 0))


def _part(d):
    return pl.BlockSpec((B, d), lambda i: (i, 0))


def _part2(d):
    return pl.BlockSpec((2, B, d), lambda i: (0, i, 0))


def _full(r, c):
    return pl.BlockSpec((r, c), lambda i: (0, 0))


def kernel(x, edge_index, W1l, b1, W1r, W2l, b2, W2r, Wlin, blin, W4l, b4, W4r):
    pad_src = jnp.arange(E_PAD - E, dtype=jnp.int32) % N
    src = jnp.concatenate([edge_index[0], pad_src]).reshape(NCHUNKS, CH)
    pad_dst = N + jnp.arange(E_PAD - E, dtype=jnp.int32) % (N_PAD - N)
    dst = jnp.concatenate([edge_index[1], pad_dst]).reshape(NCHUNKS, CH)
    z128 = jnp.zeros((N_PAD, 128), jnp.float32)
    ones128 = jnp.ones((CH, 128), jnp.float32)

    s1p, cntp = _agg_cnt(x, src, dst, z128, ones128)

    h1, y2 = pl.pallas_call(
        _tc_a,
        grid=(GRID,),
        in_specs=[_part(128), _part(128), _rows(128), _full(128, 256),
                  _full(1, 256), _full(128, 256), _full(256, 128)],
        out_specs=[_rows(256), _rows(128)],
        out_shape=[jax.ShapeDtypeStruct((N, 256), jnp.float32),
                   jax.ShapeDtypeStruct((N, 128), jnp.float32)],
    )(s1p, cntp, x, W1l.T, b1.reshape(1, -1), W1r.T, W2l.T)

    (s2p,) = _agg128(y2, src, dst, z128)

    h3 = pl.pallas_call(
        _tc_b,
        grid=(GRID,),
        in_specs=[_part2(128), _part(128), _rows(256), _full(256, 128),
                  _full(1, 128), _full(128, 128), _full(1, 128)],
        out_specs=_rows(128),
        out_shape=jax.ShapeDtypeStruct((N, 128), jnp.float32),
    )(s2p, cntp, h1, W2r.T, b2.reshape(1, -1), Wlin.T, blin.reshape(1, -1))

    (s4p,) = _agg128(h3, src, dst, z128)

    out = pl.pallas_call(
        _tc_c,
        grid=(GRID,),
        in_specs=[_part2(128), _part(128), _rows(128), _full(128, 64),
                  _full(1, 64), _full(128, 64)],
        out_specs=_rows(64),
        out_shape=jax.ShapeDtypeStruct((N, 64), jnp.float32),
    )(s4p, cntp, h3, W4l.T, b4.reshape(1, -1), W4r.T)

    return out


# speedup vs baseline: 4.0664x; 1.2175x over previous
import functools

import jax
import jax.numpy as jnp
from jax import lax
from jax.experimental import pallas as pl
from jax.experimental.pallas import tpu as pltpu
from jax.experimental.pallas import tpu_sc as plsc

N = 10000
E = 320000

NC = 2
NS = 16
NW = NC * NS
CH = 128
CG = 8
NCHUNKS = 2560
TPT = NCHUNKS // NS
E_PAD = NCHUNKS * CH
N_PAD = 10112
RPT = N_PAD // NS


_MESH = plsc.VectorSubcoreMesh(
    core_axis_name="c", subcore_axis_name="s", num_cores=NC)


def _gather_loop(table, src_r, dst_r, srcv, dstv, bufs, sems, acc, chunk0,
                 ngroups, cntv=None):
    ones_v = jnp.ones((16,), jnp.float32)

    def group(g, carry):
        g0 = pl.multiple_of(chunk0 + g * CG, CG)
        pltpu.sync_copy(src_r.at[pl.ds(g0, CG)], srcv)
        pltpu.sync_copy(dst_r.at[pl.ds(g0, CG)], dstv)
        h = pltpu.async_copy(table.at[srcv.at[0]], bufs[0], sems[0])
        for j in range(CG):
            if j + 1 < CG:
                h_next = pltpu.async_copy(
                    table.at[srcv.at[j + 1]], bufs[(j + 1) % 2],
                    sems[(j + 1) % 2])
            if cntv is not None:
                for k in range(CH // 16):
                    idx = dstv[j, pl.ds(k * 16, 16)]
                    plsc.addupdate_scatter(cntv, [idx], ones_v)
            h.wait()
            pltpu.sync_copy(bufs[j % 2], acc.at[dstv.at[j]], add=True)
            if j + 1 < CG:
                h = h_next
        return carry

    lax.fori_loop(0, ngroups, group, 0)


def _make_agg(with_cnt):
    out_type = [jax.ShapeDtypeStruct((NC, N_PAD, 128), jnp.float32)]
    scratch = [
        pltpu.VMEM((CG, CH), jnp.int32),
        pltpu.VMEM((CG, CH), jnp.int32),
        pltpu.VMEM((CH, 128), jnp.float32),
        pltpu.VMEM((CH, 128), jnp.float32),
        pltpu.VMEM_SHARED((N_PAD, 128), jnp.float32),
        pltpu.SemaphoreType.DMA,
        pltpu.SemaphoreType.DMA,
    ]
    if with_cnt:
        out_type.append(jax.ShapeDtypeStruct((NW, N_PAD), jnp.float32))
        scratch.append(pltpu.VMEM((N_PAD,), jnp.float32))

    def common(table, src_r, dst_r, zeros, out_s, srcv, dstv, rows0, rows1,
               acc, sem0, sem1, out_c=None, cntv=None):
        c = lax.axis_index("c")
        s = lax.axis_index("s")
        r0 = s * RPT
        chunk0 = c * (NCHUNKS // NC) + s * (TPT // NC)
        pltpu.sync_copy(zeros.at[pl.ds(r0, RPT)], acc.at[pl.ds(r0, RPT)])
        if with_cnt:
            zv = jnp.zeros((16,), jnp.float32)
            for i in range(N_PAD // 16):
                cntv[pl.ds(i * 16, 16)] = zv
        plsc.subcore_barrier()
        _gather_loop(table, src_r, dst_r, srcv, dstv, (rows0, rows1),
                     (sem0, sem1), acc, chunk0, TPT // NC // CG, cntv)
        plsc.subcore_barrier()
        pltpu.sync_copy(acc.at[pl.ds(r0, RPT)], out_s.at[c, pl.ds(r0, RPT)])
        if with_cnt:
            wid = s * NC + c
            pltpu.sync_copy(cntv, out_c.at[wid])

    if with_cnt:
        def body(table, src_r, dst_r, zeros, out_s, out_c,
                 srcv, dstv, rows0, rows1, acc, sem0, sem1, cntv):
            common(table, src_r, dst_r, zeros, out_s, srcv, dstv, rows0,
                   rows1, acc, sem0, sem1, out_c=out_c, cntv=cntv)
    else:
        def body(table, src_r, dst_r, zeros, out_s,
                 srcv, dstv, rows0, rows1, acc, sem0, sem1):
            common(table, src_r, dst_r, zeros, out_s, srcv, dstv, rows0,
                   rows1, acc, sem0, sem1)

    return pl.kernel(
        body, mesh=_MESH, out_type=out_type, scratch_types=scratch,
        compiler_params=pltpu.CompilerParams(
            needs_layout_passes=False) if with_cnt else None)


_agg_cnt = _make_agg(True)
_agg128 = _make_agg(False)

B = 1000
GRID = N // B


def _inv_cnt(cntp_ref):
    cnt = jnp.sum(cntp_ref[...], axis=1)[:, None]
    return 1.0 / jnp.maximum(cnt, 1.0)


def _tc_a(s1p, cntp, x, w1lt, b1, w1rt, w2lt, h1_o, y2_o):
    mean = (s1p[0] + s1p[1]) * _inv_cnt(cntp)
    h1 = jnp.maximum(
        jnp.dot(mean, w1lt[...], preferred_element_type=jnp.float32)
        + b1[...]
        + jnp.dot(x[...], w1rt[...], preferred_element_type=jnp.float32),
        0.0,
    )
    h1_o[...] = h1
    y2_o[...] = jnp.dot(h1, w2lt[...], preferred_element_type=jnp.float32)


def _tc_b(s2p, cntp, h1, w2rt, b2, wlint, blin, h3_o):
    mean2 = (s2p[0] + s2p[1]) * _inv_cnt(cntp)
    h2 = jnp.maximum(
        mean2 + b2[...]
        + jnp.dot(h1[...], w2rt[...], preferred_element_type=jnp.float32),
        0.0,
    )
    h3_o[...] = (
        jnp.dot(h2, wlint[...], preferred_element_type=jnp.float32) + blin[...]
    )


def _tc_c(s4p, cntp, h3, w4lt, b4, w4rt, out_o):
    mean4 = (s4p[0] + s4p[1]) * _inv_cnt(cntp)
    out_o[...] = (
        jnp.dot(mean4, w4lt[...], preferred_element_type=jnp.float32)
        + b4[...]
        + jnp.dot(h3[...], w4rt[...], preferred_element_type=jnp.float32)
    )


def _rows(d):
    return pl.BlockSpec((B, d), lambda i: (i, 0))


def _part(d):
    return pl.BlockSpec((B, d), lambda i: (i, 0))


def _part2(d):
    return pl.BlockSpec((2, B, d), lambda i: (0, i, 0))


def _full(r, c):
    return pl.BlockSpec((r, c), lambda i: (0, 0))


_CNT_SPEC = pl.BlockSpec((B, NW), lambda i: (i, 0))


def kernel(x, edge_index, W1l, b1, W1r, W2l, b2, W2r, Wlin, blin, W4l, b4, W4r):
    pad_src = jnp.arange(E_PAD - E, dtype=jnp.int32) % N
    src = jnp.concatenate([edge_index[0], pad_src]).reshape(NCHUNKS, CH)
    pad_dst = N + jnp.arange(E_PAD - E, dtype=jnp.int32) % (N_PAD - N)
    dst = jnp.concatenate([edge_index[1], pad_dst]).reshape(NCHUNKS, CH)
    z128 = jnp.zeros((N_PAD, 128), jnp.float32)

    s1p, cntp = _agg_cnt(x, src, dst, z128)
    cntp = cntp.T

    h1, y2 = pl.pallas_call(
        _tc_a,
        grid=(GRID,),
        in_specs=[_part2(128), _CNT_SPEC, _rows(128), _full(128, 256),
                  _full(1, 256), _full(128, 256), _full(256, 128)],
        out_specs=[_rows(256), _rows(128)],
        out_shape=[jax.ShapeDtypeStruct((N, 256), jnp.float32),
                   jax.ShapeDtypeStruct((N, 128), jnp.float32)],
    )(s1p, cntp, x, W1l.T, b1.reshape(1, -1), W1r.T, W2l.T)

    (s2p,) = _agg128(y2, src, dst, z128)

    h3 = pl.pallas_call(
        _tc_b,
        grid=(GRID,),
        in_specs=[_part2(128), _CNT_SPEC, _rows(256), _full(256, 128),
                  _full(1, 128), _full(128, 128), _full(1, 128)],
        out_specs=_rows(128),
        out_shape=jax.ShapeDtypeStruct((N, 128), jnp.float32),
    )(s2p, cntp, h1, W2r.T, b2.reshape(1, -1), Wlin.T, blin.reshape(1, -1))

    (s4p,) = _agg128(h3, src, dst, z128)

    out = pl.pallas_call(
        _tc_c,
        grid=(GRID,),
        in_specs=[_part2(128), _CNT_SPEC, _rows(128), _full(128, 64),
                  _full(1, 64), _full(128, 64)],
        out_specs=_rows(64),
        out_shape=jax.ShapeDtypeStruct((N, 64), jnp.float32),
    )(s4p, cntp, h3, W4l.T, b4.reshape(1, -1), W4r.T)

    return out
